# Initial kernel scaffold; baseline (speedup 1.0000x reference)
import jax
import jax.numpy as jnp
from jax.experimental import pallas as pl


def kernel(x, edge_index, W1l, W1r, b1, W2l, W2r, b2, Wc1, bc1, Wc2, bc2):
    raise NotImplementedError("write your pallas kernel here")


# speedup vs baseline: 7.9402x; 7.9402x over previous
import functools

import jax
import jax.numpy as jnp
from jax import lax
from jax.experimental import pallas as pl
from jax.experimental.pallas import tpu as pltpu
from jax.experimental.pallas import tpu_sc as plsc

N_TILES = 32
SUBCORES = 16
CHUNK = 128


def _sc_segment_sum(table, src3, dst3, n_pad, with_deg):
    d = table.shape[1]
    n_chunks = src3.shape[1]
    rows_per_tile = n_pad // SUBCORES

    out_type = [jax.ShapeDtypeStruct((2, n_pad, d), jnp.float32)]
    scratch = [
        pltpu.VMEM((n_chunks, CHUNK), jnp.int32),
        pltpu.VMEM((n_chunks, CHUNK), jnp.int32),
        pltpu.VMEM((CHUNK, d), jnp.float32),
        pltpu.VMEM((16, d), jnp.float32),
        pltpu.VMEM_SHARED((n_pad, d), jnp.float32),
        pltpu.SemaphoreType.DMA,
    ]
    if with_deg:
        out_type.append(jax.ShapeDtypeStruct((2, n_pad), jnp.float32))
        scratch += [
            pltpu.VMEM((CHUNK,), jnp.float32),
            pltpu.VMEM((rows_per_tile,), jnp.float32),
            pltpu.VMEM_SHARED((n_pad,), jnp.float32),
        ]

    mesh = plsc.VectorSubcoreMesh(core_axis_name="c", subcore_axis_name="s")

    def body(*refs):
        if with_deg:
            (tbl, srcs, dsts, out, deg_out,
             src_v, dst_v, rows_v, zb, acc, sem, ones_v, zrow, dacc) = refs
        else:
            (tbl, srcs, dsts, out,
             src_v, dst_v, rows_v, zb, acc, sem) = refs
        c = lax.axis_index("c")
        s = lax.axis_index("s")
        g = c * SUBCORES + s

        z16 = jnp.zeros((16,), jnp.float32)
        for i in range(16):
            for j in range(d // 16):
                zb[i, pl.ds(j * 16, 16)] = z16
        if with_deg:
            for i in range(rows_per_tile // 16):
                zrow[pl.ds(i * 16, 16)] = z16
            one16 = jnp.ones((16,), jnp.float32)
            for i in range(CHUNK // 16):
                ones_v[pl.ds(i * 16, 16)] = one16

        base = s * rows_per_tile
        for r in range(rows_per_tile // 16):
            pltpu.sync_copy(zb, acc.at[pl.ds(base + r * 16, 16)])
        if with_deg:
            pltpu.sync_copy(zrow, dacc.at[pl.ds(base, rows_per_tile)])
        plsc.subcore_barrier()

        pltpu.sync_copy(srcs.at[g], src_v)
        pltpu.sync_copy(dsts.at[g], dst_v)

        def chunk_body(j, carry):
            pltpu.async_copy(tbl.at[src_v.at[j]], rows_v, sem).wait()
            pltpu.sync_copy(rows_v, acc.at[dst_v.at[j]], add=True)
            if with_deg:
                pltpu.sync_copy(ones_v, dacc.at[dst_v.at[j]], add=True)
            return carry

        lax.fori_loop(0, n_chunks, chunk_body, 0)
        plsc.subcore_barrier()

        pltpu.sync_copy(acc.at[pl.ds(base, rows_per_tile)],
                        out.at[c, pl.ds(base, rows_per_tile)])
        if with_deg:
            pltpu.sync_copy(dacc.at[pl.ds(base, rows_per_tile)],
                            deg_out.at[c, pl.ds(base, rows_per_tile)])

    fn = pl.kernel(
        body, out_type=out_type, mesh=mesh, scratch_types=scratch,
        compiler_params=pltpu.CompilerParams(use_tc_tiling_on_sc=False))
    return fn(table, src3, dst3)


def _tc_project2(x, wlT, wrT, b, blk):
    n, k = x.shape
    m = wlT.shape[1]

    def body(x_ref, wl_ref, wr_ref, b_ref, xl_ref, xr_ref):
        xb = x_ref[...]
        xl_ref[...] = jnp.dot(xb, wl_ref[...], preferred_element_type=jnp.float32)
        xr_ref[...] = (jnp.dot(xb, wr_ref[...], preferred_element_type=jnp.float32)
                       + b_ref[...])

    grid = n // blk
    return pl.pallas_call(
        body,
        grid=(grid,),
        in_specs=[
            pl.BlockSpec((blk, k), lambda i: (i, 0)),
            pl.BlockSpec((k, m), lambda i: (0, 0)),
            pl.BlockSpec((k, m), lambda i: (0, 0)),
            pl.BlockSpec((1, m), lambda i: (0, 0)),
        ],
        out_specs=[
            pl.BlockSpec((blk, m), lambda i: (i, 0)),
            pl.BlockSpec((blk, m), lambda i: (i, 0)),
        ],
        out_shape=[
            jax.ShapeDtypeStruct((n, m), jnp.float32),
            jax.ShapeDtypeStruct((n, m), jnp.float32),
        ],
    )(x, wlT, wrT, b)


def _tc_layer2(p, d3, xr, w2lT, w2rT, b2, blk):
    n, m = xr.shape
    m2 = w2lT.shape[1]

    def body(p_ref, d_ref, xr_ref, wl_ref, wr_ref, b_ref, xl2_ref, xr2_ref):
        p_blk = p_ref[...]
        agg = p_blk[0] + p_blk[1]
        dg = d_ref[...]
        deg = dg[0] + dg[1]
        r = 1.0 / jnp.maximum(deg, 1.0)
        h = jnp.maximum(agg * r + xr_ref[...], 0.0)
        xl2_ref[...] = jnp.dot(h, wl_ref[...], preferred_element_type=jnp.float32)
        xr2_ref[...] = (jnp.dot(h, wr_ref[...], preferred_element_type=jnp.float32)
                        + b_ref[...])

    grid = n // blk
    return pl.pallas_call(
        body,
        grid=(grid,),
        in_specs=[
            pl.BlockSpec((2, blk, m), lambda i: (0, i, 0)),
            pl.BlockSpec((2, blk, 1), lambda i: (0, i, 0)),
            pl.BlockSpec((blk, m), lambda i: (i, 0)),
            pl.BlockSpec((m, m2), lambda i: (0, 0)),
            pl.BlockSpec((m, m2), lambda i: (0, 0)),
            pl.BlockSpec((1, m2), lambda i: (0, 0)),
        ],
        out_specs=[
            pl.BlockSpec((blk, m2), lambda i: (i, 0)),
            pl.BlockSpec((blk, m2), lambda i: (i, 0)),
        ],
        out_shape=[
            jax.ShapeDtypeStruct((n, m2), jnp.float32),
            jax.ShapeDtypeStruct((n, m2), jnp.float32),
        ],
    )(p, d3, xr, w2lT, w2rT, b2)


def _tc_head(q, d3, xr2, wc1T, bc1, wc2T, bc2, blk):
    n, m = xr2.shape
    k1 = wc1T.shape[1]
    k2 = wc2T.shape[1]

    def body(q_ref, d_ref, xr_ref, w1_ref, b1_ref, w2_ref, b2_ref, o_ref):
        q_blk = q_ref[...]
        agg = q_blk[0] + q_blk[1]
        dg = d_ref[...]
        deg = dg[0] + dg[1]
        r = 1.0 / jnp.maximum(deg, 1.0)
        h2 = jnp.maximum(agg * r + xr_ref[...], 0.0)
        t = jnp.maximum(
            jnp.dot(h2, w1_ref[...], preferred_element_type=jnp.float32)
            + b1_ref[...], 0.0)
        o_ref[...] = (jnp.dot(t, w2_ref[...], preferred_element_type=jnp.float32)
                      + b2_ref[...])

    grid = n // blk
    return pl.pallas_call(
        body,
        grid=(grid,),
        in_specs=[
            pl.BlockSpec((2, blk, m), lambda i: (0, i, 0)),
            pl.BlockSpec((2, blk, 1), lambda i: (0, i, 0)),
            pl.BlockSpec((blk, m), lambda i: (i, 0)),
            pl.BlockSpec((m, k1), lambda i: (0, 0)),
            pl.BlockSpec((1, k1), lambda i: (0, 0)),
            pl.BlockSpec((k1, k2), lambda i: (0, 0)),
            pl.BlockSpec((1, k2), lambda i: (0, 0)),
        ],
        out_specs=pl.BlockSpec((blk, k2), lambda i: (i, 0)),
        out_shape=jax.ShapeDtypeStruct((n, k2), jnp.float32),
    )(q, d3, xr2, wc1T, bc1, wc2T, bc2)


def kernel(x, edge_index, W1l, W1r, b1, W2l, W2r, b2, Wc1, bc1, Wc2, bc2):
    n, d_in = x.shape
    e = edge_index.shape[1]
    hid = W1l.shape[0]
    hid2 = W2l.shape[0]

    n_pad = ((n // (SUBCORES * 16)) + 1) * SUBCORES * 16 if n % (SUBCORES * 16) else n
    per_tile = -(-e // N_TILES)
    n_chunks = -(-per_tile // CHUNK)
    e_pad = N_TILES * n_chunks * CHUNK

    src = edge_index[0].astype(jnp.int32)
    dst = edge_index[1].astype(jnp.int32)
    pad = e_pad - e
    src3 = jnp.concatenate([src, jnp.zeros((pad,), jnp.int32)]
                           ).reshape(N_TILES, n_chunks, CHUNK)
    dst3 = jnp.concatenate([dst, jnp.full((pad,), n, jnp.int32)]
                           ).reshape(N_TILES, n_chunks, CHUNK)

    blk = 400 if n % 400 == 0 else 8 * (n // 8)

    xl, xr1 = _tc_project2(x, W1l.T, W1r.T, b1.reshape(1, -1), blk)
    p1, degp = _sc_segment_sum(xl, src3, dst3, n_pad, with_deg=True)
    d3 = degp.reshape(2, n_pad, 1)

    xl2, xr2 = _tc_layer2(p1, d3, xr1, W2l.T, W2r.T, b2.reshape(1, -1), blk)
    p2 = _sc_segment_sum(xl2, src3, dst3, n_pad, with_deg=False)[0]

    return _tc_head(p2, d3, xr2, Wc1.T, bc1.reshape(1, -1), Wc2.T,
                    bc2.reshape(1, -1), blk)
